# baseline (speedup 1.0000x reference)
import functools

import jax
import jax.numpy as jnp
from jax import lax
from jax.experimental import pallas as pl
from jax.experimental.pallas import tpu as pltpu
from jax.experimental.pallas import tpu_sc as plsc

EPSILON = 1e-6
BATCH = 1024
NUM_VARS = 64
VAR_SIZE = 256
NUM_ANCHORS = 16
LANES = 16
NUM_WORKERS = 32
ROWS_PER_WORKER = BATCH // NUM_WORKERS
NBUF = 4
A = NUM_ANCHORS

_mesh = plsc.VectorSubcoreMesh(core_axis_name="c", subcore_axis_name="s")


@functools.partial(
    pl.kernel,
    out_type=jax.ShapeDtypeStruct((BATCH, NUM_VARS, VAR_SIZE), jnp.float32),
    mesh=_mesh,
    scratch_types=[
        pltpu.VMEM((NBUF, 3 * A, VAR_SIZE), jnp.float32),
        pltpu.VMEM((NBUF, A, VAR_SIZE), jnp.float32),
        pltpu.VMEM_SHARED((16, A, VAR_SIZE), jnp.float32),
        pltpu.SemaphoreType.DMA,
        pltpu.SemaphoreType.DMA,
        pltpu.SemaphoreType.DMA,
        pltpu.SemaphoreType.DMA,
        pltpu.SemaphoreType.DMA,
        pltpu.SemaphoreType.DMA,
        pltpu.SemaphoreType.DMA,
        pltpu.SemaphoreType.DMA,
        pltpu.SemaphoreType.DMA,
        pltpu.SemaphoreType.DMA,
    ],
)
def _sc_grad_optim(preds_hbm, gt_hbm, out_hbm, pbuf, gbuf, sbuf, *sems):
    sin = sems[:NBUF]
    sout = sems[NBUF : 2 * NBUF]
    ssin = sems[2 * NBUF]
    ssout = sems[2 * NBUF + 1]
    c = lax.axis_index("c")
    s = lax.axis_index("s")
    wid = s * 2 + c
    base = wid * ROWS_PER_WORKER

    def start_in(slot, row):
        pltpu.make_async_copy(
            preds_hbm.at[row, pl.ds(0, 2 * A)], pbuf.at[slot, pl.ds(0, 2 * A)], sin[slot]
        ).start()
        pltpu.make_async_copy(
            preds_hbm.at[row, pl.ds(3 * A, A)], pbuf.at[slot, pl.ds(2 * A, A)], sin[slot]
        ).start()
        pltpu.make_async_copy(
            gt_hbm.at[row, pl.ds(2 * A, A)], gbuf.at[slot], sin[slot]
        ).start()

    def wait_in(slot):
        pltpu.make_async_copy(
            preds_hbm.at[0, pl.ds(0, 2 * A)], pbuf.at[slot, pl.ds(0, 2 * A)], sin[slot]
        ).wait()
        pltpu.make_async_copy(
            preds_hbm.at[0, pl.ds(3 * A, A)], pbuf.at[slot, pl.ds(2 * A, A)], sin[slot]
        ).wait()
        pltpu.make_async_copy(
            gt_hbm.at[0, pl.ds(2 * A, A)], gbuf.at[slot], sin[slot]
        ).wait()

    def start_out_tail(slot, row):
        pltpu.make_async_copy(
            pbuf.at[slot, pl.ds(2 * A, A)], out_hbm.at[row, pl.ds(3 * A, A)], sout[slot]
        ).start()

    def start_out_head(slot, row):
        pltpu.make_async_copy(
            pbuf.at[slot, pl.ds(0, 2 * A)], out_hbm.at[row, pl.ds(0, 2 * A)], sout[slot]
        ).start()

    def wait_out(slot):
        pltpu.make_async_copy(
            pbuf.at[slot, pl.ds(2 * A, A)], out_hbm.at[0, pl.ds(3 * A, A)], sout[slot]
        ).wait()
        pltpu.make_async_copy(
            pbuf.at[slot, pl.ds(0, 2 * A)], out_hbm.at[0, pl.ds(0, 2 * A)], sout[slot]
        ).wait()

    def start_sp_in(row):
        pltpu.make_async_copy(
            preds_hbm.at[row, pl.ds(2 * A, A)], sbuf.at[s], ssin
        ).start()

    def wait_sp_in():
        pltpu.make_async_copy(
            preds_hbm.at[0, pl.ds(2 * A, A)], sbuf.at[s], ssin
        ).wait()

    def start_sp_out(row):
        pltpu.make_async_copy(
            sbuf.at[s], out_hbm.at[row, pl.ds(2 * A, A)], ssout
        ).start()

    def wait_sp_out():
        pltpu.make_async_copy(
            sbuf.at[s], out_hbm.at[0, pl.ds(2 * A, A)], ssout
        ).wait()

    def compute(slot):
        def per_anchor(a, carry):
            for j in range(VAR_SIZE // LANES):
                off = j * LANES
                g = gbuf[slot, a, pl.ds(off, LANES)]
                av = pbuf[slot, a, pl.ds(off, LANES)]
                m = jnp.maximum(
                    pbuf[slot, a + A, pl.ds(off, LANES)] + EPSILON,
                    pbuf[slot, a + 2 * A, pl.ds(off, LANES)] - EPSILON,
                ) - g
                pbuf[slot, a, pl.ds(off, LANES)] = jnp.maximum(av, m)
            return carry

        lax.fori_loop(0, NUM_ANCHORS, per_anchor, 0)

    for k in range(NBUF):
        start_in(k, base + k)

    def group(g, carry):
        r = base + NBUF * g
        for k in range(NBUF):
            if k == 0:
                @pl.when(g >= 1)
                def _():
                    wait_sp_out()
            else:
                wait_sp_out()
            start_sp_in(r + k)
            refill = (k + 2) % NBUF
            if k < 2:
                @pl.when(g >= 1)
                def _(refill=refill, row=r + k + 2):
                    wait_out(refill)
                    start_in(refill, row)
            else:
                @pl.when(g < ROWS_PER_WORKER // NBUF - 1)
                def _(refill=refill, row=r + k + 2):
                    wait_out(refill)
                    start_in(refill, row)
            wait_in(k)
            start_out_tail(k, r + k)
            compute(k)
            start_out_head(k, r + k)
            wait_sp_in()
            start_sp_out(r + k)
        return carry

    lax.fori_loop(0, ROWS_PER_WORKER // NBUF, group, 0)
    wait_sp_out()
    for k in range(NBUF):
        wait_out(k)


def kernel(preds, ground_truth):
    return _sc_grad_optim(preds, ground_truth)


# speedup vs baseline: 1.1763x; 1.1763x over previous
import functools

import jax
import jax.numpy as jnp
from jax import lax
from jax.experimental import pallas as pl
from jax.experimental.pallas import tpu as pltpu
from jax.experimental.pallas import tpu_sc as plsc

EPSILON = 1e-6
BATCH = 1024
NUM_VARS = 64
VAR_SIZE = 256
NUM_ANCHORS = 16
LANES = 16
NUM_WORKERS = 32
ROWS_PER_WORKER = BATCH // NUM_WORKERS
NBUF = 4
UNROLL = 4

_mesh = plsc.VectorSubcoreMesh(core_axis_name="c", subcore_axis_name="s")


@functools.partial(
    pl.kernel,
    out_type=jax.ShapeDtypeStruct((BATCH, NUM_VARS, VAR_SIZE), jnp.float32),
    mesh=_mesh,
    scratch_types=[
        pltpu.VMEM((NBUF, NUM_VARS, VAR_SIZE), jnp.float32),
        pltpu.VMEM((NBUF, NUM_ANCHORS, VAR_SIZE), jnp.float32),
        pltpu.SemaphoreType.DMA,
        pltpu.SemaphoreType.DMA,
        pltpu.SemaphoreType.DMA,
        pltpu.SemaphoreType.DMA,
        pltpu.SemaphoreType.DMA,
        pltpu.SemaphoreType.DMA,
        pltpu.SemaphoreType.DMA,
        pltpu.SemaphoreType.DMA,
    ],
)
def _sc_grad_optim(preds_hbm, gt_hbm, out_hbm, pbuf, gbuf, *sems):
    sin = sems[:NBUF]
    sout = sems[NBUF:]
    wid = lax.axis_index("s") * 2 + lax.axis_index("c")
    base = wid * ROWS_PER_WORKER

    def start_in(slot, row):
        pltpu.make_async_copy(preds_hbm.at[row], pbuf.at[slot], sin[slot]).start()
        pltpu.make_async_copy(
            gt_hbm.at[row, pl.ds(2 * NUM_ANCHORS, NUM_ANCHORS)], gbuf.at[slot], sin[slot]
        ).start()

    def wait_in_p(slot):
        pltpu.make_async_copy(preds_hbm.at[0], pbuf.at[slot], sin[slot]).wait()

    def wait_in_g(slot):
        pltpu.make_async_copy(
            gt_hbm.at[0, pl.ds(2 * NUM_ANCHORS, NUM_ANCHORS)], gbuf.at[slot], sin[slot]
        ).wait()

    def start_out_tail(slot, row):
        pltpu.make_async_copy(
            pbuf.at[slot, pl.ds(NUM_ANCHORS, NUM_VARS - NUM_ANCHORS)],
            out_hbm.at[row, pl.ds(NUM_ANCHORS, NUM_VARS - NUM_ANCHORS)],
            sout[slot],
        ).start()

    def start_out_head(slot, row):
        pltpu.make_async_copy(
            pbuf.at[slot, pl.ds(0, NUM_ANCHORS)],
            out_hbm.at[row, pl.ds(0, NUM_ANCHORS)],
            sout[slot],
        ).start()

    def wait_out(slot):
        pltpu.make_async_copy(
            pbuf.at[slot, pl.ds(NUM_ANCHORS, NUM_VARS - NUM_ANCHORS)],
            out_hbm.at[0, pl.ds(NUM_ANCHORS, NUM_VARS - NUM_ANCHORS)],
            sout[slot],
        ).wait()
        pltpu.make_async_copy(
            pbuf.at[slot, pl.ds(0, NUM_ANCHORS)],
            out_hbm.at[0, pl.ds(0, NUM_ANCHORS)],
            sout[slot],
        ).wait()

    def compute(slot):
        def per_anchor(a, carry):
            for j in range(VAR_SIZE // LANES):
                off = j * LANES
                g = gbuf[slot, a, pl.ds(off, LANES)]
                av = pbuf[slot, a, pl.ds(off, LANES)]
                m1 = (pbuf[slot, a + 16, pl.ds(off, LANES)] + EPSILON) - g
                m2 = (pbuf[slot, a + 48, pl.ds(off, LANES)] - EPSILON) - g
                pbuf[slot, a, pl.ds(off, LANES)] = jnp.maximum(
                    jnp.maximum(av, m1), m2
                )
            return carry

        lax.fori_loop(0, NUM_ANCHORS, per_anchor, 0)

    for k in range(NBUF):
        start_in(k, base + k)

    def group(g, carry):
        r = base + NBUF * g
        for k in range(NBUF):
            wait_in_p(k)
            start_out_tail(k, r + k)
            refill = (k + 2) % NBUF
            if k < 2:
                @pl.when(g >= 1)
                def _(refill=refill, row=r + k + 2):
                    wait_out(refill)
                    start_in(refill, row)
            else:
                @pl.when(g < ROWS_PER_WORKER // NBUF - 1)
                def _(refill=refill, row=r + k + 2):
                    wait_out(refill)
                    start_in(refill, row)
            wait_in_g(k)
            compute(k)
            start_out_head(k, r + k)
        return carry

    lax.fori_loop(0, ROWS_PER_WORKER // NBUF, group, 0)
    for k in range(NBUF):
        wait_out(k)


def kernel(preds, ground_truth):
    return _sc_grad_optim(preds, ground_truth)


# speedup vs baseline: 1.1817x; 1.0046x over previous
import functools

import jax
import jax.numpy as jnp
from jax import lax
from jax.experimental import pallas as pl
from jax.experimental.pallas import tpu as pltpu
from jax.experimental.pallas import tpu_sc as plsc

EPSILON = 1e-6
BATCH = 1024
NUM_VARS = 64
VAR_SIZE = 256
NUM_ANCHORS = 16
LANES = 16
NUM_WORKERS = 32
ROWS_PER_WORKER = BATCH // NUM_WORKERS
NBUF = 4
UNROLL = 4

_mesh = plsc.VectorSubcoreMesh(core_axis_name="c", subcore_axis_name="s")


@functools.partial(
    pl.kernel,
    out_type=jax.ShapeDtypeStruct((BATCH, NUM_VARS, VAR_SIZE), jnp.float32),
    mesh=_mesh,
    scratch_types=[
        pltpu.VMEM((NBUF, NUM_VARS, VAR_SIZE), jnp.float32),
        pltpu.VMEM((NBUF, NUM_ANCHORS, VAR_SIZE), jnp.float32),
        pltpu.SemaphoreType.DMA,
        pltpu.SemaphoreType.DMA,
        pltpu.SemaphoreType.DMA,
        pltpu.SemaphoreType.DMA,
        pltpu.SemaphoreType.DMA,
        pltpu.SemaphoreType.DMA,
        pltpu.SemaphoreType.DMA,
        pltpu.SemaphoreType.DMA,
    ],
)
def _sc_grad_optim(preds_hbm, gt_hbm, out_hbm, pbuf, gbuf, *sems):
    sin = sems[:NBUF]
    sout = sems[NBUF:]
    wid = lax.axis_index("s") * 2 + lax.axis_index("c")
    base = wid * ROWS_PER_WORKER

    def start_in(slot, row):
        pltpu.make_async_copy(preds_hbm.at[row], pbuf.at[slot], sin[slot]).start()
        pltpu.make_async_copy(
            gt_hbm.at[row, pl.ds(2 * NUM_ANCHORS, NUM_ANCHORS)], gbuf.at[slot], sin[slot]
        ).start()

    def wait_in_p(slot):
        pltpu.make_async_copy(preds_hbm.at[0], pbuf.at[slot], sin[slot]).wait()

    def wait_in_g(slot):
        pltpu.make_async_copy(
            gt_hbm.at[0, pl.ds(2 * NUM_ANCHORS, NUM_ANCHORS)], gbuf.at[slot], sin[slot]
        ).wait()

    def start_out_tail(slot, row):
        pltpu.make_async_copy(
            pbuf.at[slot, pl.ds(NUM_ANCHORS, NUM_VARS - NUM_ANCHORS)],
            out_hbm.at[row, pl.ds(NUM_ANCHORS, NUM_VARS - NUM_ANCHORS)],
            sout[slot],
        ).start()

    def start_out_head(slot, row):
        pltpu.make_async_copy(
            pbuf.at[slot, pl.ds(0, NUM_ANCHORS)],
            out_hbm.at[row, pl.ds(0, NUM_ANCHORS)],
            sout[slot],
        ).start()

    def wait_out(slot):
        pltpu.make_async_copy(
            pbuf.at[slot, pl.ds(NUM_ANCHORS, NUM_VARS - NUM_ANCHORS)],
            out_hbm.at[0, pl.ds(NUM_ANCHORS, NUM_VARS - NUM_ANCHORS)],
            sout[slot],
        ).wait()
        pltpu.make_async_copy(
            pbuf.at[slot, pl.ds(0, NUM_ANCHORS)],
            out_hbm.at[0, pl.ds(0, NUM_ANCHORS)],
            sout[slot],
        ).wait()

    def compute(slot):
        def per_anchor(a, carry):
            for j in range(VAR_SIZE // LANES):
                off = j * LANES
                g = gbuf[slot, a, pl.ds(off, LANES)]
                av = pbuf[slot, a, pl.ds(off, LANES)]
                m = jnp.maximum(
                    pbuf[slot, a + 16, pl.ds(off, LANES)] + EPSILON,
                    pbuf[slot, a + 48, pl.ds(off, LANES)] - EPSILON,
                ) - g
                pbuf[slot, a, pl.ds(off, LANES)] = jnp.maximum(av, m)
            return carry

        lax.fori_loop(0, NUM_ANCHORS, per_anchor, 0)

    for k in range(NBUF):
        start_in(k, base + k)

    def group(g, carry):
        r = base + NBUF * g
        for k in range(NBUF):
            wait_in_p(k)
            start_out_tail(k, r + k)
            refill = (k + 2) % NBUF
            if k < 2:
                @pl.when(g >= 1)
                def _(refill=refill, row=r + k + 2):
                    wait_out(refill)
                    start_in(refill, row)
            else:
                @pl.when(g < ROWS_PER_WORKER // NBUF - 1)
                def _(refill=refill, row=r + k + 2):
                    wait_out(refill)
                    start_in(refill, row)
            wait_in_g(k)
            compute(k)
            start_out_head(k, r + k)
        return carry

    lax.fori_loop(0, ROWS_PER_WORKER // NBUF, group, 0)
    for k in range(NBUF):
        wait_out(k)


def kernel(preds, ground_truth):
    return _sc_grad_optim(preds, ground_truth)
